# baseline (speedup 1.0000x reference)
import functools

import jax
import jax.numpy as jnp
from jax import lax
from jax.experimental import pallas as pl
from jax.experimental.pallas import tpu as pltpu
from jax.experimental.pallas import tpu_sc as plsc

IMG_H, IMG_W = 256, 704
NCAM = 6
Hf, Wf = 64, 176
HW = Hf * Wf
GPX = NCAM * HW
NPTS = 100000
NPAD = 100352
ROWS = NPAD // 128
KEYTOT = NCAM * NPAD
KPT = KEYTOT // 16
KCH = 6272
NV_B = KCH // 16
TBL = 29568
OUTCH = GPX // 16
NV_O = OUTCH // 16
SENT = 63


def _project_body(xr, yr, zr, pr, outr):
    cam = pl.program_id(1)
    (m00, m01, m02, m03, m10, m11, m12, m13,
     m20, m21, m22, m23, fx, fy, cx, cy, lo, hi) = [
        pr[0, cam, j] for j in range(18)]
    xw = xr[0]
    yw = yr[0]
    zw = zr[0]
    px = m00 * xw + m01 * yw + m02 * zw + m03
    py = m10 * xw + m11 * yw + m12 * zw + m13
    pz = m20 * xw + m21 * yw + m22 * zw + m23
    zc = jnp.maximum(pz, 1e-6)
    u = fx * (px / zc) + cx
    v = fy * (py / zc) + cy
    m = (pz > 0.1) & (u >= 0.0) & (u <= IMG_W - 1.0) \
        & (v >= 0.0) & (v <= IMG_H - 1.0)
    uf = (u * 0.25).astype(jnp.int32)
    vf = (v * 0.25).astype(jnp.int32)
    dcl = jnp.clip(pz, lo, hi)
    binv = (dcl - 1.5).astype(jnp.int32)
    binv = binv - (binv.astype(jnp.float32) + 1.5 >= dcl).astype(jnp.int32)
    binv = jnp.clip(binv, 0, 47)
    camoff = cam * (HW * 64)
    key = jnp.where(m, vf * (Wf * 64) + uf * 64 + binv + camoff,
                    camoff + SENT)
    outr[0, 0] = key


_project = pl.pallas_call(
    _project_body,
    grid=(2, NCAM),
    in_specs=[
        pl.BlockSpec((1, ROWS, 128), lambda b, n: (b, 0, 0)),
        pl.BlockSpec((1, ROWS, 128), lambda b, n: (b, 0, 0)),
        pl.BlockSpec((1, ROWS, 128), lambda b, n: (b, 0, 0)),
        pl.BlockSpec((1, NCAM, 18), lambda b, n: (b, 0, 0),
                     memory_space=pltpu.SMEM),
    ],
    out_specs=pl.BlockSpec((1, 1, ROWS, 128), lambda b, n: (b, n, 0, 0)),
    out_shape=jax.ShapeDtypeStruct((2, NCAM, ROWS, 128), jnp.int32),
    compiler_params=pltpu.CompilerParams(
        dimension_semantics=("parallel", "arbitrary")),
)


def _tbl_base(tile):
    cam_lo = (tile * KPT) // NPAD
    return (cam_lo * HW) // OUTCH * OUTCH


def _sc_body(keys, out_hbm, kb0, kb1, table, table2, win, acc, outv,
             sem0, sem1, spt):
    c = lax.axis_index("c")
    s = lax.axis_index("s")
    kbs = (kb0, kb1)
    sems = (sem0, sem1)

    sent16 = jnp.full((16,), SENT, jnp.int32)

    def ibody(i, _):
        for k in range(4):
            table[pl.ds(i * 64 + k * 16, 16)] = sent16
            table2[pl.ds(i * 64 + k * 16, 16)] = sent16
        return 0

    lax.fori_loop(0, TBL // 64, ibody, 0)

    cam_lo = (s * KPT) // NPAD
    base = (cam_lo * HW) // OUTCH * OUTCH
    basev = jnp.full((16,), 0, jnp.int32) + base
    lane = lax.iota(jnp.int32, 16)
    prev_ix = jnp.maximum(lane - 1, 0)

    def mk_bbody(buf):
        def bbody(i, _):
            for k, tbl in ((0, table), (1, table2)):
                kv = buf[pl.ds(i * 32 + k * 16, 16)]
                skv, _ = plsc.sort_key_val(kv, kv)
                locv = jnp.right_shift(skv, 6) - basev
                bv = skv & SENT
                prev = lax.gather(
                    locv, prev_ix[:, None],
                    dimension_numbers=lax.GatherDimensionNumbers(
                        offset_dims=(), collapsed_slice_dims=(0,),
                        start_index_map=(0,)),
                    slice_sizes=(1,),
                    mode=lax.GatherScatterMode.PROMISE_IN_BOUNDS)
                first = (locv != prev) | (lane == 0)
                cur = plsc.load_gather(tbl, [locv])
                need = first & (bv < cur)
                plsc.store_scatter(tbl, [locv], bv, mask=need)
            return 0
        return bbody

    h = pltpu.async_copy(keys.at[pl.ds(c * KEYTOT + s * KPT, KCH)], kb0, sem0)
    for j in range(KPT // KCH):
        buf = kbs[j % 2]
        h.wait()
        if j < KPT // KCH - 1:
            h = pltpu.async_copy(
                keys.at[pl.ds(c * KEYTOT + s * KPT + (j + 1) * KCH, KCH)],
                kbs[(j + 1) % 2], sems[(j + 1) % 2])
        lax.fori_loop(0, NV_B // 2, mk_bbody(buf), 0)

    def fbody(i, _):
        for k in range(4):
            o = i * 64 + k * 16
            table[pl.ds(o, 16)] = jnp.minimum(table[pl.ds(o, 16)],
                                              table2[pl.ds(o, 16)])
        return 0

    lax.fori_loop(0, TBL // 64, fbody, 0)
    pltpu.sync_copy(table, spt.at[s])
    plsc.subcore_barrier()

    def zbody(i, _):
        acc[pl.ds(i * 16, 16)] = sent16
        return 0

    lax.fori_loop(0, NV_O, zbody, 0)

    g0 = s * OUTCH
    for t2 in range(16):
        b2 = _tbl_base(t2)
        valid = (g0 >= b2) & (g0 + OUTCH <= b2 + TBL)

        @pl.when(valid)
        def _(t2=t2, b2=b2):
            pltpu.sync_copy(spt.at[t2, pl.ds(g0 - b2, OUTCH)], win)

            def mbody(i, _):
                acc[pl.ds(i * 16, 16)] = jnp.minimum(
                    acc[pl.ds(i * 16, 16)], win[pl.ds(i * 16, 16)])
                return 0

            lax.fori_loop(0, NV_O, mbody, 0)

    def obody(i, _):
        a = acc[pl.ds(i * 16, 16)]
        outv[pl.ds(i * 16, 16)] = jnp.where(a >= 48, -1, a)
        return 0

    lax.fori_loop(0, NV_O, obody, 0)
    pltpu.sync_copy(outv, out_hbm.at[pl.ds(c * GPX + g0, OUTCH)])


_labeler = functools.partial(
    pl.kernel,
    mesh=plsc.VectorSubcoreMesh(core_axis_name="c", subcore_axis_name="s"),
    out_type=jax.ShapeDtypeStruct((2 * GPX,), jnp.int32),
    compiler_params=pltpu.CompilerParams(needs_layout_passes=False),
    scratch_types=[
        pltpu.VMEM((KCH,), jnp.int32),
        pltpu.VMEM((KCH,), jnp.int32),
        pltpu.VMEM((TBL,), jnp.int32),
        pltpu.VMEM((TBL,), jnp.int32),
        pltpu.VMEM((OUTCH,), jnp.int32),
        pltpu.VMEM((OUTCH,), jnp.int32),
        pltpu.VMEM((OUTCH,), jnp.int32),
        pltpu.SemaphoreType.DMA,
        pltpu.SemaphoreType.DMA,
        pltpu.VMEM_SHARED((16, TBL), jnp.int32),
    ],
)(_sc_body)


def kernel(points_ego, intrinsics, cam2ego, feat_hw):
    B = points_ego.shape[0]
    ego2cam = jnp.linalg.inv(cam2ego)

    pad = NPAD - NPTS
    xyz = jnp.pad(points_ego[..., :3], ((0, 0), (0, pad), (0, 0)),
                  constant_values=float('nan'))
    xyz = xyz.astype(jnp.bfloat16).astype(jnp.float32)
    xs = xyz[..., 0].reshape(B, ROWS, 128)
    ys = xyz[..., 1].reshape(B, ROWS, 128)
    zs = xyz[..., 2].reshape(B, ROWS, 128)

    dv = 2.0 + jnp.arange(48, dtype=jnp.float32)
    step = dv[1] - dv[0]
    edges = jnp.concatenate([dv[:1] - step / 2.0, dv + step / 2.0])
    lo = edges[0] + 0.001
    hi = edges[-1] - 0.001

    mat = ego2cam[:, :, :3, :].reshape(B, NCAM, 12)
    mat = mat.astype(jnp.bfloat16).astype(jnp.float32)
    fx = intrinsics[:, :, 0, 0][..., None]
    fy = intrinsics[:, :, 1, 1][..., None]
    cx = intrinsics[:, :, 0, 2][..., None]
    cy = intrinsics[:, :, 1, 2][..., None]
    ones = jnp.ones((B, NCAM, 1), jnp.float32)
    par = jnp.concatenate([mat, fx, fy, cx, cy, lo * ones, hi * ones], axis=-1)

    keys = _project(xs, ys, zs, par).reshape(-1)
    out = _labeler(keys)
    return out.reshape(B, NCAM, Hf, Wf).astype(jnp.int64)


# speedup vs baseline: 1.0093x; 1.0093x over previous
import functools

import jax
import jax.numpy as jnp
from jax import lax
from jax.experimental import pallas as pl
from jax.experimental.pallas import tpu as pltpu
from jax.experimental.pallas import tpu_sc as plsc

IMG_H, IMG_W = 256, 704
NCAM = 6
Hf, Wf = 64, 176
HW = Hf * Wf
GPX = NCAM * HW
NPTS = 100000
NPAD = 100352
ROWS = NPAD // 128
KEYTOT = NCAM * NPAD
KPT = KEYTOT // 16
KCH = 6272
NV_B = KCH // 16
TBL = 29568
OUTCH = GPX // 16
NV_O = OUTCH // 16
SENT = 63


def _project_body(xr, yr, zr, pr, outr):
    xw = _bf16r(xr[0])
    yw = _bf16r(yr[0])
    zw = _bf16r(zr[0])
    for cam in range(NCAM):
        (m00, m01, m02, m03, m10, m11, m12, m13,
         m20, m21, m22, m23, fx, fy, cx, cy, lo, hi) = [
            pr[0, cam, j] for j in range(18)]
        px = m00 * xw + m01 * yw + m02 * zw + m03
        py = m10 * xw + m11 * yw + m12 * zw + m13
        pz = m20 * xw + m21 * yw + m22 * zw + m23
        zc = jnp.maximum(pz, 1e-6)
        u = fx * (px / zc) + cx
        v = fy * (py / zc) + cy
        m = (pz > 0.1) & (u >= 0.0) & (u <= IMG_W - 1.0) \
            & (v >= 0.0) & (v <= IMG_H - 1.0)
        uf = (u * 0.25).astype(jnp.int32)
        vf = (v * 0.25).astype(jnp.int32)
        dcl = jnp.clip(pz, lo, hi)
        binv = (dcl - 1.5).astype(jnp.int32)
        binv = binv - (binv.astype(jnp.float32) + 1.5 >= dcl).astype(jnp.int32)
        binv = jnp.clip(binv, 0, 47)
        camoff = cam * (HW * 64)
        key = jnp.where(m, vf * (Wf * 64) + uf * 64 + binv + camoff,
                        camoff + SENT)
        outr[0, cam] = key


def _bf16r(x):
    return x.astype(jnp.bfloat16).astype(jnp.float32)


_project = pl.pallas_call(
    _project_body,
    grid=(2,),
    in_specs=[
        pl.BlockSpec((1, ROWS, 128), lambda b: (b, 0, 0)),
        pl.BlockSpec((1, ROWS, 128), lambda b: (b, 0, 0)),
        pl.BlockSpec((1, ROWS, 128), lambda b: (b, 0, 0)),
        pl.BlockSpec((1, NCAM, 18), lambda b: (b, 0, 0),
                     memory_space=pltpu.SMEM),
    ],
    out_specs=pl.BlockSpec((1, NCAM, ROWS, 128), lambda b: (b, 0, 0, 0)),
    out_shape=jax.ShapeDtypeStruct((2, NCAM, ROWS, 128), jnp.int32),
    compiler_params=pltpu.CompilerParams(
        dimension_semantics=("parallel",)),
)


def _tbl_base(tile):
    cam_lo = (tile * KPT) // NPAD
    return (cam_lo * HW) // OUTCH * OUTCH


def _sc_body(keys, out_hbm, kb0, kb1, table, win, acc, outv,
             sem0, sem1, spt):
    c = lax.axis_index("c")
    s = lax.axis_index("s")
    kbs = (kb0, kb1)
    sems = (sem0, sem1)

    sent16 = jnp.full((16,), SENT, jnp.int32)

    def ibody(i, _):
        for k in range(4):
            table[pl.ds(i * 64 + k * 16, 16)] = sent16
        return 0

    lax.fori_loop(0, TBL // 64, ibody, 0)

    cam_lo = (s * KPT) // NPAD
    base = (cam_lo * HW) // OUTCH * OUTCH
    basev = jnp.full((16,), 0, jnp.int32) + base
    lane = lax.iota(jnp.int32, 16)
    prev_ix = jnp.maximum(lane - 1, 0)

    def mk_bbody(buf):
        def bbody(i, _):
            kv = buf[pl.ds(i * 16, 16)]
            skv, _ = plsc.sort_key_val(kv, kv)
            locv = jnp.right_shift(skv, 6) - basev
            bv = skv & SENT
            prev = lax.gather(
                locv, prev_ix[:, None],
                dimension_numbers=lax.GatherDimensionNumbers(
                    offset_dims=(), collapsed_slice_dims=(0,),
                    start_index_map=(0,)),
                slice_sizes=(1,),
                mode=lax.GatherScatterMode.PROMISE_IN_BOUNDS)
            first = (locv != prev) | (lane == 0)
            cur = plsc.load_gather(table, [locv])
            need = first & (bv < cur)
            plsc.store_scatter(table, [locv], bv, mask=need)
            return 0
        return bbody

    h = pltpu.async_copy(keys.at[pl.ds(c * KEYTOT + s * KPT, KCH)], kb0, sem0)
    for j in range(KPT // KCH):
        buf = kbs[j % 2]
        h.wait()
        if j < KPT // KCH - 1:
            h = pltpu.async_copy(
                keys.at[pl.ds(c * KEYTOT + s * KPT + (j + 1) * KCH, KCH)],
                kbs[(j + 1) % 2], sems[(j + 1) % 2])
        lax.fori_loop(0, NV_B, mk_bbody(buf), 0)

    pltpu.sync_copy(table, spt.at[s])
    plsc.subcore_barrier()

    def zbody(i, _):
        acc[pl.ds(i * 16, 16)] = sent16
        return 0

    lax.fori_loop(0, NV_O, zbody, 0)

    g0 = s * OUTCH
    for t2 in range(16):
        b2 = _tbl_base(t2)
        valid = (g0 >= b2) & (g0 + OUTCH <= b2 + TBL)

        @pl.when(valid)
        def _(t2=t2, b2=b2):
            pltpu.sync_copy(spt.at[t2, pl.ds(g0 - b2, OUTCH)], win)

            def mbody(i, _):
                acc[pl.ds(i * 16, 16)] = jnp.minimum(
                    acc[pl.ds(i * 16, 16)], win[pl.ds(i * 16, 16)])
                return 0

            lax.fori_loop(0, NV_O, mbody, 0)

    def obody(i, _):
        a = acc[pl.ds(i * 16, 16)]
        outv[pl.ds(i * 16, 16)] = jnp.where(a >= 48, -1, a)
        return 0

    lax.fori_loop(0, NV_O, obody, 0)
    pltpu.sync_copy(outv, out_hbm.at[pl.ds(c * GPX + g0, OUTCH)])


_labeler = functools.partial(
    pl.kernel,
    mesh=plsc.VectorSubcoreMesh(core_axis_name="c", subcore_axis_name="s"),
    out_type=jax.ShapeDtypeStruct((2 * GPX,), jnp.int32),
    compiler_params=pltpu.CompilerParams(needs_layout_passes=False),
    scratch_types=[
        pltpu.VMEM((KCH,), jnp.int32),
        pltpu.VMEM((KCH,), jnp.int32),
        pltpu.VMEM((TBL,), jnp.int32),
        pltpu.VMEM((OUTCH,), jnp.int32),
        pltpu.VMEM((OUTCH,), jnp.int32),
        pltpu.VMEM((OUTCH,), jnp.int32),
        pltpu.SemaphoreType.DMA,
        pltpu.SemaphoreType.DMA,
        pltpu.VMEM_SHARED((16, TBL), jnp.int32),
    ],
)(_sc_body)


def kernel(points_ego, intrinsics, cam2ego, feat_hw):
    B = points_ego.shape[0]
    ego2cam = jnp.linalg.inv(cam2ego)

    pad = NPAD - NPTS
    xyz = jnp.pad(points_ego[..., :3], ((0, 0), (0, pad), (0, 0)),
                  constant_values=float('nan'))
    xs = xyz[..., 0].reshape(B, ROWS, 128)
    ys = xyz[..., 1].reshape(B, ROWS, 128)
    zs = xyz[..., 2].reshape(B, ROWS, 128)

    dv = 2.0 + jnp.arange(48, dtype=jnp.float32)
    step = dv[1] - dv[0]
    edges = jnp.concatenate([dv[:1] - step / 2.0, dv + step / 2.0])
    lo = edges[0] + 0.001
    hi = edges[-1] - 0.001

    mat = ego2cam[:, :, :3, :].reshape(B, NCAM, 12)
    mat = mat.astype(jnp.bfloat16).astype(jnp.float32)
    fx = intrinsics[:, :, 0, 0][..., None]
    fy = intrinsics[:, :, 1, 1][..., None]
    cx = intrinsics[:, :, 0, 2][..., None]
    cy = intrinsics[:, :, 1, 2][..., None]
    ones = jnp.ones((B, NCAM, 1), jnp.float32)
    par = jnp.concatenate([mat, fx, fy, cx, cy, lo * ones, hi * ones], axis=-1)

    keys = _project(xs, ys, zs, par).reshape(-1)
    out = _labeler(keys)
    return out.reshape(B, NCAM, Hf, Wf).astype(jnp.int64)
